# baseline (speedup 1.0000x reference)
import functools

import jax
import jax.numpy as jnp
from jax import lax
from jax.experimental import pallas as pl
from jax.experimental.pallas import tpu as pltpu
from jax.experimental.pallas import tpu_sc as plsc

_BATCH = 4
_SEQ = 2048
_D = 1024
_TOTAL = _BATCH * _SEQ * _D
_POS_TOTAL = _SEQ * _D
_NC = 2
_NS = 16
_NW = _NC * _NS
_SPAN = _TOTAL // _NW
_CHUNK = 32768
_NCHUNK = _SPAN // _CHUNK


@functools.partial(
    pl.kernel,
    mesh=plsc.VectorSubcoreMesh(core_axis_name="c", subcore_axis_name="s"),
    out_type=jax.ShapeDtypeStruct((_TOTAL,), jnp.float32),
    scratch_types=[
        pltpu.VMEM((_CHUNK,), jnp.float32),
        pltpu.VMEM((_CHUNK,), jnp.float32),
    ],
)
def _sc_add(x_hbm, pos_hbm, out_hbm, bufx, bufp):
    wid = lax.axis_index("s") * _NC + lax.axis_index("c")
    base = wid * _SPAN
    pos_base = base % _POS_TOTAL

    def chunk_body(c, carry):
        off = pl.multiple_of(base + c * _CHUNK, _CHUNK)
        poff = pl.multiple_of(pos_base + c * _CHUNK, _CHUNK)
        pltpu.sync_copy(x_hbm.at[pl.ds(off, _CHUNK)], bufx)
        pltpu.sync_copy(pos_hbm.at[pl.ds(poff, _CHUNK)], bufp)

        def add_body(i, c2):
            s = pl.ds(i * 16, 16)
            bufx[s] = bufx[s] + bufp[s]
            return c2

        lax.fori_loop(0, _CHUNK // 16, add_body, 0)
        pltpu.sync_copy(bufx, out_hbm.at[pl.ds(off, _CHUNK)])
        return carry

    lax.fori_loop(0, _NCHUNK, chunk_body, 0)


def kernel(x, pos_embedding):
    seq = x.shape[1]
    xf = x.reshape(-1)
    pf = pos_embedding[:seq].reshape(-1)
    out = _sc_add(xf, pf)
    return out.reshape(x.shape)


# speedup vs baseline: 1.3615x; 1.3615x over previous
import functools

import jax
import jax.numpy as jnp
from jax import lax
from jax.experimental import pallas as pl
from jax.experimental.pallas import tpu as pltpu
from jax.experimental.pallas import tpu_sc as plsc

_BATCH = 4
_SEQ = 2048
_D = 1024
_TOTAL = _BATCH * _SEQ * _D
_POS_TOTAL = _SEQ * _D
_NC = 2
_NS = 16
_NW = _NC * _NS
_SPAN = _TOTAL // _NW
_CHUNK = 32768
_NCHUNK = _SPAN // _CHUNK


@functools.partial(
    pl.kernel,
    mesh=plsc.VectorSubcoreMesh(core_axis_name="c", subcore_axis_name="s"),
    out_type=jax.ShapeDtypeStruct((_TOTAL,), jnp.float32),
    scratch_types=[
        pltpu.VMEM((_CHUNK,), jnp.float32),
        pltpu.VMEM((_CHUNK,), jnp.float32),
    ],
)
def _sc_add(x_hbm, pos_hbm, out_hbm, bufx, bufp):
    wid = lax.axis_index("s") * _NC + lax.axis_index("c")
    base = wid * _SPAN
    pos_base = base % _POS_TOTAL

    def chunk_body(c, carry):
        off = pl.multiple_of(base + c * _CHUNK, _CHUNK)
        poff = pl.multiple_of(pos_base + c * _CHUNK, _CHUNK)
        pltpu.sync_copy(x_hbm.at[pl.ds(off, _CHUNK)], bufx)
        pltpu.sync_copy(pos_hbm.at[pl.ds(poff, _CHUNK)], bufp)

        @plsc.parallel_loop(0, _CHUNK // 16, unroll=8)
        def add_body(i):
            s = pl.ds(i * 16, 16)
            bufx[s] = bufx[s] + bufp[s]

        pltpu.sync_copy(bufx, out_hbm.at[pl.ds(off, _CHUNK)])
        return carry

    lax.fori_loop(0, _NCHUNK, chunk_body, 0)


def kernel(x, pos_embedding):
    seq = x.shape[1]
    xf = x.reshape(-1)
    pf = pos_embedding[:seq].reshape(-1)
    out = _sc_add(xf, pf)
    return out.reshape(x.shape)


# speedup vs baseline: 1.5834x; 1.1630x over previous
import functools

import jax
import jax.numpy as jnp
from jax import lax
from jax.experimental import pallas as pl
from jax.experimental.pallas import tpu as pltpu
from jax.experimental.pallas import tpu_sc as plsc

_BATCH = 4
_SEQ = 2048
_D = 1024
_TOTAL = _BATCH * _SEQ * _D
_POS_TOTAL = _SEQ * _D
_NC = 2
_NS = 16
_NW = _NC * _NS
_SPAN = _TOTAL // _NW
_CHUNK = 16384
_NCHUNK = _SPAN // _CHUNK


@functools.partial(
    pl.kernel,
    mesh=plsc.VectorSubcoreMesh(core_axis_name="c", subcore_axis_name="s"),
    out_type=jax.ShapeDtypeStruct((_TOTAL,), jnp.float32),
    scratch_types=[
        pltpu.VMEM((_CHUNK,), jnp.float32),
        pltpu.VMEM((_CHUNK,), jnp.float32),
        pltpu.VMEM((_CHUNK,), jnp.float32),
        pltpu.VMEM((_CHUNK,), jnp.float32),
        pltpu.SemaphoreType.DMA,
        pltpu.SemaphoreType.DMA,
        pltpu.SemaphoreType.DMA,
        pltpu.SemaphoreType.DMA,
        pltpu.SemaphoreType.DMA,
        pltpu.SemaphoreType.DMA,
    ],
)
def _sc_add(x_hbm, pos_hbm, out_hbm, bufx0, bufx1, bufp0, bufp1,
            semx0, semx1, semp0, semp1, semo0, semo1):
    wid = lax.axis_index("s") * _NC + lax.axis_index("c")
    base = wid * _SPAN
    pos_base = base % _POS_TOTAL
    bufx = (bufx0, bufx1)
    bufp = (bufp0, bufp1)
    semx = (semx0, semx1)
    semp = (semp0, semp1)
    semo = (semo0, semo1)

    def start_in(g, slot):
        off = pl.multiple_of(base + g * _CHUNK, _CHUNK)
        poff = pl.multiple_of(pos_base + g * _CHUNK, _CHUNK)
        hx = pltpu.make_async_copy(
            x_hbm.at[pl.ds(off, _CHUNK)], bufx[slot], semx[slot])
        hp = pltpu.make_async_copy(
            pos_hbm.at[pl.ds(poff, _CHUNK)], bufp[slot], semp[slot])
        hx.start()
        hp.start()
        return hx, hp

    def start_out(g, slot):
        off = pl.multiple_of(base + g * _CHUNK, _CHUNK)
        ho = pltpu.make_async_copy(
            bufx[slot], out_hbm.at[pl.ds(off, _CHUNK)], semo[slot])
        ho.start()
        return ho

    in_h = {}
    out_h = {}
    for g in range(_NCHUNK + 1):
        if g < _NCHUNK:
            if g >= 2:
                out_h.pop(g - 2).wait()
            in_h[g] = start_in(g, g % 2)
        if g >= 1:
            gg = g - 1
            slot = gg % 2
            hx, hp = in_h.pop(gg)
            hx.wait()
            hp.wait()
            bx = bufx[slot]
            bp = bufp[slot]

            @plsc.parallel_loop(0, _CHUNK // 16, unroll=8)
            def add_body(i, bx=bx, bp=bp):
                s = pl.ds(i * 16, 16)
                bx[s] = bx[s] + bp[s]

            out_h[gg] = start_out(gg, slot)
    out_h.pop(_NCHUNK - 2).wait()
    out_h.pop(_NCHUNK - 1).wait()


def kernel(x, pos_embedding):
    seq = x.shape[1]
    xf = x.reshape(-1)
    pf = pos_embedding[:seq].reshape(-1)
    out = _sc_add(xf, pf)
    return out.reshape(x.shape)


# speedup vs baseline: 8.9653x; 5.6622x over previous
import jax
import jax.numpy as jnp
from jax.experimental import pallas as pl


def _add_body(x_ref, pos_ref, out_ref):
    out_ref[...] = x_ref[...] + pos_ref[...]


def kernel(x, pos_embedding):
    batch, seq, d = x.shape
    pos = pos_embedding[:seq]
    return pl.pallas_call(
        _add_body,
        grid=(batch,),
        in_specs=[
            pl.BlockSpec((1, seq, d), lambda b: (b, 0, 0)),
            pl.BlockSpec((seq, d), lambda b: (0, 0)),
        ],
        out_specs=pl.BlockSpec((1, seq, d), lambda b: (b, 0, 0)),
        out_shape=jax.ShapeDtypeStruct(x.shape, x.dtype),
    )(x, pos)
